# Initial kernel scaffold; baseline (speedup 1.0000x reference)
import jax
import jax.numpy as jnp
from jax.experimental import pallas as pl


def kernel(x, Wg, W1, b1, W2, b2):
    raise NotImplementedError("write your pallas kernel here")


# speedup vs baseline: 7.5722x; 7.5722x over previous
import functools

import jax
import jax.numpy as jnp
from jax import lax
from jax.experimental import pallas as pl
from jax.experimental.pallas import tpu as pltpu
from jax.experimental.pallas import tpu_sc as plsc

E = 16
DIM = 768
HID = 3072
NTOK = 2048
TB = 128
NBLK = NTOK // TB + E
NPAD = NBLK * TB
NW = 32
BPW = NTOK // NW


def _route_plan_body(x_ref, wg_ref, dest_ref, bexp_ref, na_ref):
    x = x_ref[...]
    wg = wg_ref[...]
    logits = jnp.dot(x, wg, preferred_element_type=jnp.float32)

    rowmax = jnp.max(logits, axis=1, keepdims=True)
    eq = logits == rowmax
    incl = (lax.broadcasted_iota(jnp.int32, (E, E), 0)
            <= lax.broadcasted_iota(jnp.int32, (E, E), 1)).astype(jnp.bfloat16)
    prefix = jnp.dot(eq.astype(jnp.bfloat16), incl,
                     preferred_element_type=jnp.float32)
    onehot = jnp.where(jnp.logical_and(eq, prefix == 1.0), 1.0, 0.0)
    oneb = onehot.astype(jnp.bfloat16)

    counts = jnp.sum(onehot, axis=0, keepdims=True)
    nb = jnp.ceil(counts * (1.0 / TB))
    ends_b = jnp.dot(nb.astype(jnp.bfloat16), incl,
                     preferred_element_type=jnp.float32)
    offsets = (ends_b - nb) * TB

    CH = 128
    NCH = NTOK // CH
    tril = (lax.broadcasted_iota(jnp.int32, (CH, CH), 1)
            < lax.broadcasted_iota(jnp.int32, (CH, CH), 0)).astype(jnp.bfloat16)
    trilc = (lax.broadcasted_iota(jnp.int32, (NCH, NCH), 1)
             < lax.broadcasted_iota(jnp.int32, (NCH, NCH), 0)).astype(jnp.bfloat16)
    chunks = [oneb[c * CH:(c + 1) * CH, :] for c in range(NCH)]
    totals = jnp.concatenate(
        [jnp.sum(onehot[c * CH:(c + 1) * CH, :], axis=0, keepdims=True)
         for c in range(NCH)], axis=0)
    choffs = jnp.dot(trilc, totals.astype(jnp.bfloat16),
                     preferred_element_type=jnp.float32)
    rank = jnp.concatenate(
        [jnp.dot(tril, chunks[c], preferred_element_type=jnp.float32)
         + choffs[c:c + 1, :] for c in range(NCH)], axis=0)

    dest = jnp.sum((rank + offsets) * onehot, axis=1, keepdims=True)
    dest_ref[...] = dest.astype(jnp.int32)

    elast = (lax.broadcasted_iota(jnp.int32, (E, 1), 0)
             == (E - 1)).astype(jnp.bfloat16)
    total_b = jnp.dot(ends_b.astype(jnp.bfloat16), elast,
                      preferred_element_type=jnp.float32)
    na = total_b.astype(jnp.int32)
    na_ref[...] = na
    iota_blk = lax.broadcasted_iota(jnp.int32, (NBLK, 1), 0)
    jt = jnp.minimum(iota_blk * TB, na * TB - TB)
    ends_tok = (ends_b * TB).astype(jnp.int32)
    bexp = jnp.sum((ends_tok <= jt).astype(jnp.int32), axis=1, keepdims=True)
    bexp_ref[...] = bexp


_route_plan = pl.pallas_call(
    _route_plan_body,
    out_shape=(
        jax.ShapeDtypeStruct((NTOK, 1), jnp.int32),
        jax.ShapeDtypeStruct((NBLK, 1), jnp.int32),
        jax.ShapeDtypeStruct((1, 1), jnp.int32),
    ),
)


_SQRT_HALF = 0.7071067811865476


def _ffn_body(meta_ref, xs_ref, w1_ref, b1_ref, w2_ref, b2_ref, ys_ref):
    i = pl.program_id(0)
    na = meta_ref[NBLK]

    @pl.when(i < na)
    def _():
        xb = xs_ref[...]
        h = jnp.dot(xb, w1_ref[0], preferred_element_type=jnp.float32)
        h = h + b1_ref[0]
        h = 0.5 * h * (1.0 + lax.erf(h * _SQRT_HALF))
        y = jnp.dot(h, w2_ref[0], preferred_element_type=jnp.float32)
        ys_ref[...] = y + b2_ref[0]


_ffn = pl.pallas_call(
    _ffn_body,
    grid_spec=pltpu.PrefetchScalarGridSpec(
        num_scalar_prefetch=1,
        grid=(NBLK,),
        in_specs=[
            pl.BlockSpec((TB, DIM), lambda i, m: (i, 0)),
            pl.BlockSpec((1, DIM, HID), lambda i, m: (m[i], 0, 0)),
            pl.BlockSpec((1, 1, HID), lambda i, m: (m[i], 0, 0)),
            pl.BlockSpec((1, HID, DIM), lambda i, m: (m[i], 0, 0)),
            pl.BlockSpec((1, 1, DIM), lambda i, m: (m[i], 0, 0)),
        ],
        out_specs=pl.BlockSpec((TB, DIM), lambda i, m: (i, 0)),
    ),
    out_shape=jax.ShapeDtypeStruct((NPAD, DIM), jnp.float32),
    compiler_params=pltpu.CompilerParams(
        dimension_semantics=("arbitrary",),
    ),
)


@functools.cache
def _sc_kernels():
    mesh = plsc.VectorSubcoreMesh(core_axis_name="c", subcore_axis_name="s")

    @functools.partial(
        pl.kernel,
        out_type=jax.ShapeDtypeStruct((NPAD, DIM), jnp.float32),
        mesh=mesh,
        scratch_types=[
            pltpu.VMEM((BPW,), jnp.int32),
            pltpu.VMEM((BPW, DIM), jnp.float32),
            pltpu.SemaphoreType.DMA,
        ],
    )
    def dispatch(x_hbm, idx_hbm, xs_hbm, idx_v, rows_v, sem):
        wid = lax.axis_index("s") * 2 + lax.axis_index("c")
        pltpu.sync_copy(idx_hbm.at[wid], idx_v)
        pltpu.sync_copy(x_hbm.at[pl.ds(wid * BPW, BPW)], rows_v)
        pltpu.async_copy(rows_v, xs_hbm.at[idx_v], sem).wait()

    @functools.partial(
        pl.kernel,
        out_type=jax.ShapeDtypeStruct((NTOK, DIM), jnp.float32),
        mesh=mesh,
        scratch_types=[
            pltpu.VMEM((BPW,), jnp.int32),
            pltpu.VMEM((BPW, DIM), jnp.float32),
            pltpu.SemaphoreType.DMA,
        ],
    )
    def combine(ys_hbm, idx_hbm, out_hbm, idx_v, rows_v, sem):
        wid = lax.axis_index("s") * 2 + lax.axis_index("c")
        pltpu.sync_copy(idx_hbm.at[wid], idx_v)
        pltpu.async_copy(ys_hbm.at[idx_v], rows_v, sem).wait()
        pltpu.sync_copy(rows_v, out_hbm.at[pl.ds(wid * BPW, BPW)])

    return dispatch, combine


def kernel(x, Wg, W1, b1, W2, b2):
    x2d = x.reshape(NTOK, DIM)
    dispatch, combine = _sc_kernels()
    dest, bexp, na = _route_plan(x2d, Wg)
    meta = jnp.concatenate([bexp.reshape(NBLK), na.reshape(1)])
    idx = dest.reshape(NW, BPW)
    xs = dispatch(x2d, idx)
    ys = _ffn(meta, xs, W1, b1.reshape(E, 1, HID), W2, b2.reshape(E, 1, DIM))
    out = combine(ys, idx)
    return out.reshape(1, NTOK, DIM)


# speedup vs baseline: 8.5963x; 1.1353x over previous
import functools

import jax
import jax.numpy as jnp
from jax import lax
from jax.experimental import pallas as pl
from jax.experimental.pallas import tpu as pltpu
from jax.experimental.pallas import tpu_sc as plsc

E = 16
DIM = 768
HID = 3072
NTOK = 2048
TB = 256
NBLK = NTOK // TB + E
NPAD = NBLK * TB
NW = 32
BPW = NTOK // NW


def _route_plan_body(x_ref, wg_ref, dest_ref, bexp_ref, na_ref):
    x = x_ref[...]
    wg = wg_ref[...]
    logits = jnp.dot(x, wg, preferred_element_type=jnp.float32)

    rowmax = jnp.max(logits, axis=1, keepdims=True)
    eq = logits == rowmax
    incl = (lax.broadcasted_iota(jnp.int32, (E, E), 0)
            <= lax.broadcasted_iota(jnp.int32, (E, E), 1)).astype(jnp.bfloat16)
    prefix = jnp.dot(eq.astype(jnp.bfloat16), incl,
                     preferred_element_type=jnp.float32)
    onehot = jnp.where(jnp.logical_and(eq, prefix == 1.0), 1.0, 0.0)
    oneb = onehot.astype(jnp.bfloat16)

    counts = jnp.sum(onehot, axis=0, keepdims=True)
    nb = jnp.ceil(counts * (1.0 / TB))
    ends_b = jnp.dot(nb.astype(jnp.bfloat16), incl,
                     preferred_element_type=jnp.float32)
    offsets = (ends_b - nb) * TB

    CH = 128
    NCH = NTOK // CH
    tril = (lax.broadcasted_iota(jnp.int32, (CH, CH), 1)
            < lax.broadcasted_iota(jnp.int32, (CH, CH), 0)).astype(jnp.bfloat16)
    trilc = (lax.broadcasted_iota(jnp.int32, (NCH, NCH), 1)
             < lax.broadcasted_iota(jnp.int32, (NCH, NCH), 0)).astype(jnp.bfloat16)
    chunks = [oneb[c * CH:(c + 1) * CH, :] for c in range(NCH)]
    totals = jnp.concatenate(
        [jnp.sum(onehot[c * CH:(c + 1) * CH, :], axis=0, keepdims=True)
         for c in range(NCH)], axis=0)
    choffs = jnp.dot(trilc, totals.astype(jnp.bfloat16),
                     preferred_element_type=jnp.float32)
    rank = jnp.concatenate(
        [jnp.dot(tril, chunks[c], preferred_element_type=jnp.float32)
         + choffs[c:c + 1, :] for c in range(NCH)], axis=0)

    dest = jnp.sum((rank + offsets) * onehot, axis=1, keepdims=True)
    dest_ref[...] = dest.astype(jnp.int32)

    elast = (lax.broadcasted_iota(jnp.int32, (E, 1), 0)
             == (E - 1)).astype(jnp.bfloat16)
    total_b = jnp.dot(ends_b.astype(jnp.bfloat16), elast,
                      preferred_element_type=jnp.float32)
    na = total_b.astype(jnp.int32)
    na_ref[...] = na
    iota_blk = lax.broadcasted_iota(jnp.int32, (NBLK, 1), 0)
    jt = jnp.minimum(iota_blk * TB, na * TB - TB)
    ends_tok = (ends_b * TB).astype(jnp.int32)
    bexp = jnp.sum((ends_tok <= jt).astype(jnp.int32), axis=1, keepdims=True)
    bexp_ref[...] = bexp


_route_plan = pl.pallas_call(
    _route_plan_body,
    out_shape=(
        jax.ShapeDtypeStruct((NTOK, 1), jnp.int32),
        jax.ShapeDtypeStruct((NBLK, 1), jnp.int32),
        jax.ShapeDtypeStruct((1, 1), jnp.int32),
    ),
)


_SQRT_HALF = 0.7071067811865476


def _ffn_body(meta_ref, xs_ref, w1_ref, b1_ref, w2_ref, b2_ref, ys_ref):
    i = pl.program_id(0)
    na = meta_ref[NBLK]

    @pl.when(i < na)
    def _():
        xb = xs_ref[...]
        h = jnp.dot(xb, w1_ref[0], preferred_element_type=jnp.float32)
        h = h + b1_ref[0]
        h = 0.5 * h * (1.0 + lax.erf(h * _SQRT_HALF))
        y = jnp.dot(h, w2_ref[0], preferred_element_type=jnp.float32)
        ys_ref[...] = y + b2_ref[0]


_ffn = pl.pallas_call(
    _ffn_body,
    grid_spec=pltpu.PrefetchScalarGridSpec(
        num_scalar_prefetch=1,
        grid=(NBLK,),
        in_specs=[
            pl.BlockSpec((TB, DIM), lambda i, m: (i, 0)),
            pl.BlockSpec((1, DIM, HID), lambda i, m: (m[i], 0, 0)),
            pl.BlockSpec((1, 1, HID), lambda i, m: (m[i], 0, 0)),
            pl.BlockSpec((1, HID, DIM), lambda i, m: (m[i], 0, 0)),
            pl.BlockSpec((1, 1, DIM), lambda i, m: (m[i], 0, 0)),
        ],
        out_specs=pl.BlockSpec((TB, DIM), lambda i, m: (i, 0)),
    ),
    out_shape=jax.ShapeDtypeStruct((NPAD, DIM), jnp.float32),
    compiler_params=pltpu.CompilerParams(
        dimension_semantics=("arbitrary",),
    ),
)


@functools.cache
def _sc_kernels():
    mesh = plsc.VectorSubcoreMesh(core_axis_name="c", subcore_axis_name="s")

    @functools.partial(
        pl.kernel,
        out_type=jax.ShapeDtypeStruct((NPAD, DIM), jnp.float32),
        mesh=mesh,
        scratch_types=[
            pltpu.VMEM((BPW,), jnp.int32),
            pltpu.VMEM((BPW, DIM), jnp.float32),
            pltpu.SemaphoreType.DMA,
        ],
    )
    def dispatch(x_hbm, idx_hbm, xs_hbm, idx_v, rows_v, sem):
        wid = lax.axis_index("s") * 2 + lax.axis_index("c")
        pltpu.sync_copy(idx_hbm.at[wid], idx_v)
        pltpu.sync_copy(x_hbm.at[pl.ds(wid * BPW, BPW)], rows_v)
        pltpu.async_copy(rows_v, xs_hbm.at[idx_v], sem).wait()

    @functools.partial(
        pl.kernel,
        out_type=jax.ShapeDtypeStruct((NTOK, DIM), jnp.float32),
        mesh=mesh,
        scratch_types=[
            pltpu.VMEM((BPW,), jnp.int32),
            pltpu.VMEM((BPW, DIM), jnp.float32),
            pltpu.SemaphoreType.DMA,
        ],
    )
    def combine(ys_hbm, idx_hbm, out_hbm, idx_v, rows_v, sem):
        wid = lax.axis_index("s") * 2 + lax.axis_index("c")
        pltpu.sync_copy(idx_hbm.at[wid], idx_v)
        pltpu.async_copy(ys_hbm.at[idx_v], rows_v, sem).wait()
        pltpu.sync_copy(rows_v, out_hbm.at[pl.ds(wid * BPW, BPW)])

    return dispatch, combine


def kernel(x, Wg, W1, b1, W2, b2):
    x2d = x.reshape(NTOK, DIM)
    dispatch, combine = _sc_kernels()
    dest, bexp, na = _route_plan(x2d, Wg)
    meta = jnp.concatenate([bexp.reshape(NBLK), na.reshape(1)])
    idx = dest.reshape(NW, BPW)
    xs = dispatch(x2d, idx)
    ys = _ffn(meta, xs, W1, b1.reshape(E, 1, HID), W2, b2.reshape(E, 1, DIM))
    out = combine(ys, idx)
    return out.reshape(1, NTOK, DIM)


# speedup vs baseline: 8.9594x; 1.0422x over previous
import functools

import jax
import jax.numpy as jnp
from jax import lax
from jax.experimental import pallas as pl
from jax.experimental.pallas import tpu as pltpu
from jax.experimental.pallas import tpu_sc as plsc

E = 16
DIM = 768
HID = 3072
NTOK = 2048
TB = 256
NBLK = NTOK // TB + E
NPAD = NBLK * TB
NW = 32
BPW = NTOK // NW


def _route_plan_body(x_ref, wg_ref, dest_ref, meta_ref):
    x = x_ref[...]
    wg = wg_ref[...]
    logits = jnp.dot(x, wg, preferred_element_type=jnp.float32)

    rowmax = jnp.max(logits, axis=1, keepdims=True)
    eq = logits == rowmax
    incl = (lax.broadcasted_iota(jnp.int32, (E, E), 0)
            <= lax.broadcasted_iota(jnp.int32, (E, E), 1)).astype(jnp.bfloat16)
    prefix = jnp.dot(eq.astype(jnp.bfloat16), incl,
                     preferred_element_type=jnp.float32)
    onehot = jnp.where(jnp.logical_and(eq, prefix == 1.0), 1.0, 0.0)
    oneb = onehot.astype(jnp.bfloat16)

    counts = jnp.sum(onehot, axis=0, keepdims=True)
    nb = jnp.ceil(counts * (1.0 / TB))
    ends_b = jnp.dot(nb.astype(jnp.bfloat16), incl,
                     preferred_element_type=jnp.float32)
    offsets = (ends_b - nb) * TB

    CH = 128
    NCH = NTOK // CH
    tril = (lax.broadcasted_iota(jnp.int32, (CH, CH), 1)
            < lax.broadcasted_iota(jnp.int32, (CH, CH), 0)).astype(jnp.bfloat16)
    trilc = (lax.broadcasted_iota(jnp.int32, (NCH, NCH), 1)
             < lax.broadcasted_iota(jnp.int32, (NCH, NCH), 0)).astype(jnp.bfloat16)
    chunks = [oneb[c * CH:(c + 1) * CH, :] for c in range(NCH)]
    totals = jnp.concatenate(
        [jnp.sum(onehot[c * CH:(c + 1) * CH, :], axis=0, keepdims=True)
         for c in range(NCH)], axis=0)
    choffs = jnp.dot(trilc, totals.astype(jnp.bfloat16),
                     preferred_element_type=jnp.float32)
    rank = jnp.concatenate(
        [jnp.dot(tril, chunks[c], preferred_element_type=jnp.float32)
         + choffs[c:c + 1, :] for c in range(NCH)], axis=0)

    dest = jnp.sum((rank + offsets) * onehot, axis=1, keepdims=True)
    dest_ref[...] = dest.astype(jnp.int32)

    elast = (lax.broadcasted_iota(jnp.int32, (E, 1), 0)
             == (E - 1)).astype(jnp.bfloat16)
    total_b = jnp.dot(ends_b.astype(jnp.bfloat16), elast,
                      preferred_element_type=jnp.float32)
    na = total_b.astype(jnp.int32)
    iota_blk = lax.broadcasted_iota(jnp.int32, (NBLK, 1), 0)
    jt = jnp.minimum(iota_blk * TB, na * TB - TB)
    ends_tok = (ends_b * TB).astype(jnp.int32)
    bexp = jnp.sum((ends_tok <= jt).astype(jnp.int32), axis=1, keepdims=True)
    meta_ref[:NBLK] = bexp
    meta_ref[NBLK:] = na


_route_plan = pl.pallas_call(
    _route_plan_body,
    out_shape=(
        jax.ShapeDtypeStruct((NTOK, 1), jnp.int32),
        jax.ShapeDtypeStruct((NBLK + 1, 1), jnp.int32),
    ),
)


_SQRT_HALF = 0.7071067811865476


def _ffn_body(meta_ref, xs_ref, w1_ref, b1_ref, w2_ref, b2_ref, ys_ref):
    i = pl.program_id(0)
    na = meta_ref[NBLK]

    @pl.when(i < na)
    def _():
        xb = xs_ref[...]
        h = jnp.dot(xb, w1_ref[0], preferred_element_type=jnp.float32)
        h = h + b1_ref[0]
        h = 0.5 * h * (1.0 + lax.erf(h * _SQRT_HALF))
        y = jnp.dot(h, w2_ref[0], preferred_element_type=jnp.float32)
        ys_ref[...] = y + b2_ref[0]


def _blk(i, m):
    return jnp.minimum(i, m[NBLK] - 1)


_ffn = pl.pallas_call(
    _ffn_body,
    grid_spec=pltpu.PrefetchScalarGridSpec(
        num_scalar_prefetch=1,
        grid=(NBLK,),
        in_specs=[
            pl.BlockSpec((TB, DIM), lambda i, m: (_blk(i, m), 0)),
            pl.BlockSpec((1, DIM, HID), lambda i, m: (m[i], 0, 0)),
            pl.BlockSpec((1, 1, HID), lambda i, m: (m[i], 0, 0)),
            pl.BlockSpec((1, HID, DIM), lambda i, m: (m[i], 0, 0)),
            pl.BlockSpec((1, 1, DIM), lambda i, m: (m[i], 0, 0)),
        ],
        out_specs=pl.BlockSpec((TB, DIM), lambda i, m: (_blk(i, m), 0)),
    ),
    out_shape=jax.ShapeDtypeStruct((NPAD, DIM), jnp.float32),
    compiler_params=pltpu.CompilerParams(
        dimension_semantics=("arbitrary",),
    ),
)


@functools.cache
def _sc_kernels():
    mesh = plsc.VectorSubcoreMesh(core_axis_name="c", subcore_axis_name="s")

    @functools.partial(
        pl.kernel,
        out_type=jax.ShapeDtypeStruct((NPAD, DIM), jnp.float32),
        mesh=mesh,
        scratch_types=[
            pltpu.VMEM((BPW,), jnp.int32),
            pltpu.VMEM((BPW, DIM), jnp.float32),
            pltpu.SemaphoreType.DMA,
        ],
    )
    def dispatch(x_hbm, idx_hbm, xs_hbm, idx_v, rows_v, sem):
        wid = lax.axis_index("s") * 2 + lax.axis_index("c")
        pltpu.sync_copy(idx_hbm.at[wid], idx_v)
        pltpu.sync_copy(x_hbm.at[pl.ds(wid * BPW, BPW)], rows_v)
        pltpu.async_copy(rows_v, xs_hbm.at[idx_v], sem).wait()

    @functools.partial(
        pl.kernel,
        out_type=jax.ShapeDtypeStruct((NTOK, DIM), jnp.float32),
        mesh=mesh,
        scratch_types=[
            pltpu.VMEM((BPW,), jnp.int32),
            pltpu.VMEM((BPW, DIM), jnp.float32),
            pltpu.SemaphoreType.DMA,
        ],
    )
    def combine(ys_hbm, idx_hbm, out_hbm, idx_v, rows_v, sem):
        wid = lax.axis_index("s") * 2 + lax.axis_index("c")
        pltpu.sync_copy(idx_hbm.at[wid], idx_v)
        pltpu.async_copy(ys_hbm.at[idx_v], rows_v, sem).wait()
        pltpu.sync_copy(rows_v, out_hbm.at[pl.ds(wid * BPW, BPW)])

    return dispatch, combine


def kernel(x, Wg, W1, b1, W2, b2):
    x2d = x.reshape(NTOK, DIM)
    dispatch, combine = _sc_kernels()
    dest, meta = _route_plan(x2d, Wg)
    idx = dest.reshape(NW, BPW)
    xs = dispatch(x2d, idx)
    ys = _ffn(meta.reshape(NBLK + 1), xs,
              W1, b1.reshape(E, 1, HID), W2, b2.reshape(E, 1, DIM))
    out = combine(ys, idx)
    return out.reshape(1, NTOK, DIM)


# speedup vs baseline: 9.1610x; 1.0225x over previous
import functools

import jax
import jax.numpy as jnp
from jax import lax
from jax.experimental import pallas as pl
from jax.experimental.pallas import tpu as pltpu
from jax.experimental.pallas import tpu_sc as plsc

E = 16
DIM = 768
HID = 3072
NTOK = 2048
TB = 256
NBLK = NTOK // TB + E
NPAD = NBLK * TB
NW = 32
BPW = NTOK // NW


def _route_plan_body(x_ref, wg_ref, dest_ref, meta_ref, xb16_ref):
    x = x_ref[...]
    wg = wg_ref[...]
    logits = jnp.dot(x, wg, preferred_element_type=jnp.float32)
    bits = lax.bitcast_convert_type(x, jnp.uint32)
    rnd = (bits + jnp.uint32(0x7FFF)
           + ((bits >> jnp.uint32(16)) & jnp.uint32(1))) >> jnp.uint32(16)
    xpack = rnd[:, :DIM // 2] | (rnd[:, DIM // 2:] << jnp.uint32(16))
    xb16_ref[...] = lax.bitcast_convert_type(xpack, jnp.int32)

    rowmax = jnp.max(logits, axis=1, keepdims=True)
    eq = logits == rowmax
    incl = (lax.broadcasted_iota(jnp.int32, (E, E), 0)
            <= lax.broadcasted_iota(jnp.int32, (E, E), 1)).astype(jnp.bfloat16)
    prefix = jnp.dot(eq.astype(jnp.bfloat16), incl,
                     preferred_element_type=jnp.float32)
    onehot = jnp.where(jnp.logical_and(eq, prefix == 1.0), 1.0, 0.0)
    oneb = onehot.astype(jnp.bfloat16)

    counts = jnp.sum(onehot, axis=0, keepdims=True)
    nb = jnp.ceil(counts * (1.0 / TB))
    ends_b = jnp.dot(nb.astype(jnp.bfloat16), incl,
                     preferred_element_type=jnp.float32)
    offsets = (ends_b - nb) * TB

    CH = 128
    NCH = NTOK // CH
    tril = (lax.broadcasted_iota(jnp.int32, (CH, CH), 1)
            < lax.broadcasted_iota(jnp.int32, (CH, CH), 0)).astype(jnp.bfloat16)
    trilc = (lax.broadcasted_iota(jnp.int32, (NCH, NCH), 1)
             < lax.broadcasted_iota(jnp.int32, (NCH, NCH), 0)).astype(jnp.bfloat16)
    chunks = [oneb[c * CH:(c + 1) * CH, :] for c in range(NCH)]
    totals = jnp.concatenate(
        [jnp.sum(onehot[c * CH:(c + 1) * CH, :], axis=0, keepdims=True)
         for c in range(NCH)], axis=0)
    choffs = jnp.dot(trilc, totals.astype(jnp.bfloat16),
                     preferred_element_type=jnp.float32)
    rank = jnp.concatenate(
        [jnp.dot(tril, chunks[c], preferred_element_type=jnp.float32)
         + choffs[c:c + 1, :] for c in range(NCH)], axis=0)

    dest = jnp.sum((rank + offsets) * onehot, axis=1, keepdims=True)
    dest_ref[...] = dest.astype(jnp.int32)

    elast = (lax.broadcasted_iota(jnp.int32, (E, 1), 0)
             == (E - 1)).astype(jnp.bfloat16)
    total_b = jnp.dot(ends_b.astype(jnp.bfloat16), elast,
                      preferred_element_type=jnp.float32)
    na = total_b.astype(jnp.int32)
    iota_blk = lax.broadcasted_iota(jnp.int32, (NBLK, 1), 0)
    jt = jnp.minimum(iota_blk * TB, na * TB - TB)
    ends_tok = (ends_b * TB).astype(jnp.int32)
    bexp = jnp.sum((ends_tok <= jt).astype(jnp.int32), axis=1, keepdims=True)
    meta_ref[:NBLK] = bexp
    meta_ref[NBLK:] = na


_route_plan = pl.pallas_call(
    _route_plan_body,
    out_shape=(
        jax.ShapeDtypeStruct((NTOK, 1), jnp.int32),
        jax.ShapeDtypeStruct((NBLK + 1, 1), jnp.int32),
        jax.ShapeDtypeStruct((NTOK, DIM // 2), jnp.int32),
    ),
)


_SQRT_HALF = 0.7071067811865476


def _ffn_body(meta_ref, xs_ref, w1_ref, b1_ref, w2_ref, b2_ref, ys_ref):
    i = pl.program_id(0)
    na = meta_ref[NBLK]

    @pl.when(i < na)
    def _():
        xp = xs_ref[...]
        xlo = lax.bitcast_convert_type(xp << 16, jnp.float32)
        xhi = lax.bitcast_convert_type(xp & jnp.int32(-65536), jnp.float32)
        xb = jnp.concatenate([xlo, xhi], axis=1)
        h = jnp.dot(xb, w1_ref[0], preferred_element_type=jnp.float32)
        h = h + b1_ref[0]
        h = 0.5 * h * (1.0 + lax.erf(h * _SQRT_HALF))
        y = jnp.dot(h, w2_ref[0], preferred_element_type=jnp.float32)
        ys_ref[...] = y + b2_ref[0]


def _blk(i, m):
    return jnp.minimum(i, m[NBLK] - 1)


_ffn = pl.pallas_call(
    _ffn_body,
    grid_spec=pltpu.PrefetchScalarGridSpec(
        num_scalar_prefetch=1,
        grid=(NBLK,),
        in_specs=[
            pl.BlockSpec((TB, DIM // 2), lambda i, m: (_blk(i, m), 0)),
            pl.BlockSpec((1, DIM, HID), lambda i, m: (m[i], 0, 0)),
            pl.BlockSpec((1, 1, HID), lambda i, m: (m[i], 0, 0)),
            pl.BlockSpec((1, HID, DIM), lambda i, m: (m[i], 0, 0)),
            pl.BlockSpec((1, 1, DIM), lambda i, m: (m[i], 0, 0)),
        ],
        out_specs=pl.BlockSpec((TB, DIM), lambda i, m: (_blk(i, m), 0)),
    ),
    out_shape=jax.ShapeDtypeStruct((NPAD, DIM), jnp.float32),
    compiler_params=pltpu.CompilerParams(
        dimension_semantics=("arbitrary",),
    ),
)


@functools.cache
def _sc_kernels():
    mesh = plsc.VectorSubcoreMesh(core_axis_name="c", subcore_axis_name="s")

    @functools.partial(
        pl.kernel,
        out_type=jax.ShapeDtypeStruct((NPAD, DIM // 2), jnp.int32),
        mesh=mesh,
        scratch_types=[
            pltpu.VMEM((BPW,), jnp.int32),
            pltpu.VMEM((BPW, DIM // 2), jnp.int32),
            pltpu.SemaphoreType.DMA,
        ],
    )
    def dispatch(x_hbm, idx_hbm, xs_hbm, idx_v, rows_v, sem):
        wid = lax.axis_index("s") * 2 + lax.axis_index("c")
        pltpu.sync_copy(idx_hbm.at[wid], idx_v)
        pltpu.sync_copy(x_hbm.at[pl.ds(wid * BPW, BPW)], rows_v)
        pltpu.async_copy(rows_v, xs_hbm.at[idx_v], sem).wait()

    @functools.partial(
        pl.kernel,
        out_type=jax.ShapeDtypeStruct((NTOK, DIM), jnp.float32),
        mesh=mesh,
        scratch_types=[
            pltpu.VMEM((BPW,), jnp.int32),
            pltpu.VMEM((BPW, DIM), jnp.float32),
            pltpu.SemaphoreType.DMA,
        ],
    )
    def combine(ys_hbm, idx_hbm, out_hbm, idx_v, rows_v, sem):
        wid = lax.axis_index("s") * 2 + lax.axis_index("c")
        pltpu.sync_copy(idx_hbm.at[wid], idx_v)
        pltpu.async_copy(ys_hbm.at[idx_v], rows_v, sem).wait()
        pltpu.sync_copy(rows_v, out_hbm.at[pl.ds(wid * BPW, BPW)])

    return dispatch, combine


def kernel(x, Wg, W1, b1, W2, b2):
    x2d = x.reshape(NTOK, DIM)
    dispatch, combine = _sc_kernels()
    dest, meta, xb16 = _route_plan(x2d, Wg)
    idx = dest.reshape(NW, BPW)
    xs = dispatch(xb16, idx)
    ys = _ffn(meta.reshape(NBLK + 1), xs,
              W1, b1.reshape(E, 1, HID), W2, b2.reshape(E, 1, DIM))
    out = combine(ys, idx)
    return out.reshape(1, NTOK, DIM)


# speedup vs baseline: 9.3361x; 1.0191x over previous
import functools

import jax
import jax.numpy as jnp
from jax import lax
from jax.experimental import pallas as pl
from jax.experimental.pallas import tpu as pltpu
from jax.experimental.pallas import tpu_sc as plsc

E = 16
DIM = 768
HID = 3072
NTOK = 2048
TB = 256
NBLK = NTOK // TB + E
NPAD = NBLK * TB
NW = 32
BPW = NTOK // NW


def _route_plan_body(x_ref, wg_ref, dest_ref, meta_ref, xb16_ref):
    x = x_ref[0]
    wg = wg_ref[...]
    logits = jnp.dot(x, wg, preferred_element_type=jnp.float32)
    bits = lax.bitcast_convert_type(x, jnp.uint32)
    rnd = (bits + jnp.uint32(0x7FFF)
           + ((bits >> jnp.uint32(16)) & jnp.uint32(1))) >> jnp.uint32(16)
    xpack = rnd[:, :DIM // 2] | (rnd[:, DIM // 2:] << jnp.uint32(16))
    xb16_ref[...] = lax.bitcast_convert_type(xpack, jnp.int32)

    rowmax = jnp.max(logits, axis=1, keepdims=True)
    eq = logits == rowmax
    incl = (lax.broadcasted_iota(jnp.int32, (E, E), 0)
            <= lax.broadcasted_iota(jnp.int32, (E, E), 1)).astype(jnp.bfloat16)
    prefix = jnp.dot(eq.astype(jnp.bfloat16), incl,
                     preferred_element_type=jnp.float32)
    onehot = jnp.where(jnp.logical_and(eq, prefix == 1.0), 1.0, 0.0)
    oneb = onehot.astype(jnp.bfloat16)

    counts = jnp.sum(onehot, axis=0, keepdims=True)
    nb = jnp.ceil(counts * (1.0 / TB))
    ends_b = jnp.dot(nb.astype(jnp.bfloat16), incl,
                     preferred_element_type=jnp.float32)
    offsets = (ends_b - nb) * TB

    CH = 128
    NCH = NTOK // CH
    tril = (lax.broadcasted_iota(jnp.int32, (CH, CH), 1)
            < lax.broadcasted_iota(jnp.int32, (CH, CH), 0)).astype(jnp.bfloat16)
    trilc = (lax.broadcasted_iota(jnp.int32, (NCH, NCH), 1)
             < lax.broadcasted_iota(jnp.int32, (NCH, NCH), 0)).astype(jnp.bfloat16)
    chunks = [oneb[c * CH:(c + 1) * CH, :] for c in range(NCH)]
    totals = jnp.concatenate(
        [jnp.sum(onehot[c * CH:(c + 1) * CH, :], axis=0, keepdims=True)
         for c in range(NCH)], axis=0)
    choffs = jnp.dot(trilc, totals.astype(jnp.bfloat16),
                     preferred_element_type=jnp.float32)
    rank = jnp.concatenate(
        [jnp.dot(tril, chunks[c], preferred_element_type=jnp.float32)
         + choffs[c:c + 1, :] for c in range(NCH)], axis=0)

    dest = jnp.sum((rank + offsets) * onehot, axis=1)
    dest_ref[...] = dest.astype(jnp.int32).reshape(NW, BPW)

    elast = (lax.broadcasted_iota(jnp.int32, (E, 1), 0)
             == (E - 1)).astype(jnp.bfloat16)
    total_b = jnp.dot(ends_b.astype(jnp.bfloat16), elast,
                      preferred_element_type=jnp.float32)
    na = total_b.astype(jnp.int32)
    iota_blk = lax.broadcasted_iota(jnp.int32, (NBLK, 1), 0)
    jt = jnp.minimum(iota_blk * TB, na * TB - TB)
    ends_tok = (ends_b * TB).astype(jnp.int32)
    bexp = jnp.sum((ends_tok <= jt).astype(jnp.int32), axis=1, keepdims=True)
    meta_ref[:NBLK] = bexp
    meta_ref[NBLK:] = na


_route_plan = pl.pallas_call(
    _route_plan_body,
    out_shape=(
        jax.ShapeDtypeStruct((NW, BPW), jnp.int32),
        jax.ShapeDtypeStruct((NBLK + 1, 1), jnp.int32),
        jax.ShapeDtypeStruct((NTOK, DIM // 2), jnp.int32),
    ),
)


_SQRT_HALF = 0.7071067811865476


def _ffn_body(meta_ref, xs_ref, w1_ref, b1_ref, w2_ref, b2_ref, ys_ref):
    i = pl.program_id(0)
    na = meta_ref[NBLK]

    @pl.when(i < na)
    def _():
        xp = xs_ref[...]
        xlo = lax.bitcast_convert_type(xp << 16, jnp.float32)
        xhi = lax.bitcast_convert_type(xp & jnp.int32(-65536), jnp.float32)
        xb = jnp.concatenate([xlo, xhi], axis=1)
        h = jnp.dot(xb, w1_ref[0], preferred_element_type=jnp.float32)
        h = h + b1_ref[0]
        h = 0.5 * h * (1.0 + lax.erf(h * _SQRT_HALF))
        y = jnp.dot(h, w2_ref[0], preferred_element_type=jnp.float32)
        ys_ref[...] = y + b2_ref[0]


def _blk(i, m):
    return jnp.minimum(i, m[NBLK] - 1)


_ffn = pl.pallas_call(
    _ffn_body,
    grid_spec=pltpu.PrefetchScalarGridSpec(
        num_scalar_prefetch=1,
        grid=(NBLK,),
        in_specs=[
            pl.BlockSpec((TB, DIM // 2), lambda i, m: (_blk(i, m), 0)),
            pl.BlockSpec((1, DIM, HID), lambda i, m: (m[i], 0, 0)),
            pl.BlockSpec((1, 1, HID), lambda i, m: (m[i], 0, 0)),
            pl.BlockSpec((1, HID, DIM), lambda i, m: (m[i], 0, 0)),
            pl.BlockSpec((1, 1, DIM), lambda i, m: (m[i], 0, 0)),
        ],
        out_specs=pl.BlockSpec((TB, DIM), lambda i, m: (_blk(i, m), 0)),
    ),
    out_shape=jax.ShapeDtypeStruct((NPAD, DIM), jnp.float32),
    compiler_params=pltpu.CompilerParams(
        dimension_semantics=("arbitrary",),
    ),
)


@functools.cache
def _sc_kernels():
    mesh = plsc.VectorSubcoreMesh(core_axis_name="c", subcore_axis_name="s")

    @functools.partial(
        pl.kernel,
        out_type=jax.ShapeDtypeStruct((NPAD, DIM // 2), jnp.int32),
        mesh=mesh,
        scratch_types=[
            pltpu.VMEM((BPW,), jnp.int32),
            pltpu.VMEM((BPW, DIM // 2), jnp.int32),
            pltpu.SemaphoreType.DMA,
        ],
    )
    def dispatch(x_hbm, idx_hbm, xs_hbm, idx_v, rows_v, sem):
        wid = lax.axis_index("s") * 2 + lax.axis_index("c")
        c1 = pltpu.async_copy(idx_hbm.at[wid], idx_v, sem)
        c2 = pltpu.async_copy(x_hbm.at[pl.ds(wid * BPW, BPW)], rows_v, sem)
        c1.wait()
        c2.wait()
        pltpu.async_copy(rows_v, xs_hbm.at[idx_v], sem).wait()

    @functools.partial(
        pl.kernel,
        out_type=jax.ShapeDtypeStruct((1, NTOK, DIM), jnp.float32),
        mesh=mesh,
        scratch_types=[
            pltpu.VMEM((BPW,), jnp.int32),
            pltpu.VMEM((BPW, DIM), jnp.float32),
            pltpu.SemaphoreType.DMA,
        ],
    )
    def combine(ys_hbm, idx_hbm, out_hbm, idx_v, rows_v, sem):
        wid = lax.axis_index("s") * 2 + lax.axis_index("c")
        pltpu.sync_copy(idx_hbm.at[wid], idx_v)
        pltpu.async_copy(ys_hbm.at[idx_v], rows_v, sem).wait()
        pltpu.sync_copy(rows_v, out_hbm.at[0, pl.ds(wid * BPW, BPW)])

    return dispatch, combine


def kernel(x, Wg, W1, b1, W2, b2):
    dispatch, combine = _sc_kernels()
    idx, meta, xb16 = _route_plan(x, Wg)
    xs = dispatch(xb16, idx)
    ys = _ffn(meta.reshape(NBLK + 1), xs,
              W1, b1.reshape(E, 1, HID), W2, b2.reshape(E, 1, DIM))
    return combine(ys, idx)
